# baseline (speedup 1.0000x reference)
import functools

import jax
import jax.numpy as jnp
from jax import lax
from jax.experimental import pallas as pl
from jax.experimental.pallas import tpu as pltpu
from jax.experimental.pallas import tpu_sc as plsc

_T, _BCOL = 200, 4096
_V, _D = 1_000_000, 32

_info = plsc.get_sparse_core_info()
_NC, _NS = _info.num_cores, _info.num_subcores
_NW = _NC * _NS
_NTILES = (_T // 8) * (_BCOL // 128)
_TPW = _NTILES // _NW
_NQ = 4 * _TPW

_mesh = plsc.VectorSubcoreMesh(core_axis_name="c", subcore_axis_name="s")


@functools.partial(
    pl.kernel,
    mesh=_mesh,
    out_type=jax.ShapeDtypeStruct((_T, _D, _BCOL), jnp.float32),
    scratch_types=[
        pltpu.VMEM((8, 128), jnp.int32),
        pltpu.VMEM((8, 128), jnp.int32),
        pltpu.VMEM((256,), jnp.int32),
        pltpu.VMEM((256,), jnp.int32),
        pltpu.VMEM((256,), jnp.int32),
        pltpu.VMEM((256,), jnp.int32),
        pltpu.VMEM((256, 128), jnp.float32),
        pltpu.VMEM((256, 128), jnp.float32),
        pltpu.VMEM((1, 32, 128), jnp.float32),
        pltpu.VMEM((1, 32, 128), jnp.float32),
        pltpu.VMEM((1, 32, 128), jnp.float32),
        pltpu.VMEM((1, 32, 128), jnp.float32),
        pltpu.SemaphoreType.DMA,
        pltpu.SemaphoreType.DMA,
        pltpu.SemaphoreType.DMA,
    ],
    compiler_params=pltpu.CompilerParams(
        use_tc_tiling_on_sc=True, needs_layout_passes=False
    ),
)
def _emb_lookup(
    idx_hbm,
    table_hbm,
    out_hbm,
    idxt0,
    idxt1,
    idg0,
    idg1,
    sub0,
    sub1,
    rows0,
    rows1,
    ob00,
    ob01,
    ob10,
    ob11,
    isem,
    gsem,
    osem,
):
    wid = lax.axis_index("s") * _NC + lax.axis_index("c")
    base = wid * _NQ
    iota16 = lax.iota(jnp.int32, 16)
    zero16 = iota16 * 0
    cvec0 = iota16
    cvec1 = iota16 + 16
    idxt = (idxt0, idxt1)
    idg = (idg0, idg1)
    sub = (sub0, sub1)
    rows = (rows0, rows1)
    ob = ((ob00, ob01), (ob10, ob11))

    def locate(j):
        k = j >> 2
        h2 = j & 3
        R = k >> 5
        C = k & 31
        return R, C, 8 * R + 2 * h2

    def issue_idx(j, p):
        R, C, _ = locate(j)
        pltpu.async_copy(
            idx_hbm.at[
                pl.ds(pl.multiple_of(8 * R, 8), 8),
                pl.ds(pl.multiple_of(128 * C, 128), 128),
            ],
            idxt[p],
            isem,
        )

    def wait_idx(p):
        pltpu.make_async_copy(
            idx_hbm.at[pl.ds(0, 8), pl.ds(0, 128)], idxt[p], isem
        ).wait()

    def build_gather_list(p, j):
        h2 = j & 3
        for u in range(16):
            r2 = 2 * h2 + (u >> 3)
            v = idxt[p][r2, pl.ds((u & 7) * 16, 16)]
            idg[p][pl.ds(u * 16, 16)] = v >> 2
            sub[p][pl.ds(u * 16, 16)] = (v & 3) << 5

    def issue_gather(p):
        pltpu.async_copy(table_hbm.at[idg[p]], rows[p], gsem)

    def wait_gather(p):
        pltpu.make_async_copy(table_hbm.at[pl.ds(0, 256)], rows[p], gsem).wait()

    def drain_out(p):
        for t2 in range(2):
            for Rd in range(4):
                pltpu.make_async_copy(
                    out_hbm.at[pl.ds(0, 1), pl.ds(0, 8), pl.ds(0, 128)],
                    ob[p][t2].at[pl.ds(0, 1), pl.ds(8 * Rd, 8)],
                    osem,
                ).wait()

    def transpose_and_store(p, j):
        R, C, t0 = locate(j)
        for t2 in range(2):
            qb = t2 * 128
            for tc0 in range(128):
                tcv = tc0 + iota16
                if tc0 > 112:
                    tcv = tcv & 127
                rvec = qb + tcv
                sv = plsc.load_gather(sub[p], [rvec])
                for cvec in (cvec0, cvec1):
                    v = plsc.load_gather(rows[p], [rvec, sv + cvec])
                    plsc.store_scatter(ob[p][t2], [zero16, cvec, tcv], v)
        for t2 in range(2):
            for Rd in range(4):
                pltpu.async_copy(
                    ob[p][t2].at[pl.ds(0, 1), pl.ds(8 * Rd, 8)],
                    out_hbm.at[
                        pl.ds(t0 + t2, 1),
                        pl.ds(8 * Rd, 8),
                        pl.ds(pl.multiple_of(128 * C, 128), 128),
                    ],
                    osem,
                )

    issue_idx(base, 0)
    wait_idx(0)
    build_gather_list(0, base)
    issue_gather(0)
    issue_idx(base + 1, 1)

    def body(i2, carry):
        j = base + 2 * i2
        @pl.when(i2 > 0)
        def _():
            drain_out(0)

        wait_gather(0)
        wait_idx(1)
        build_gather_list(1, j + 1)
        issue_gather(1)

        @pl.when(i2 < _NQ // 2 - 1)
        def _():
            issue_idx(j + 2, 0)

        transpose_and_store(0, j)

        @pl.when(i2 > 0)
        def _():
            drain_out(1)

        wait_gather(1)

        @pl.when(i2 < _NQ // 2 - 1)
        def _():
            wait_idx(0)
            build_gather_list(0, j + 2)
            issue_gather(0)
            issue_idx(j + 3, 1)

        transpose_and_store(1, j + 1)
        return carry

    lax.fori_loop(0, _NQ // 2, body, 0)
    drain_out(0)
    drain_out(1)


def kernel(inputs, embedding_weight):
    out3 = _emb_lookup(
        inputs.astype(jnp.int32), embedding_weight.reshape(_V // 4, 4 * _D)
    )
    return out3.transpose(0, 2, 1)


# speedup vs baseline: 1.4571x; 1.4571x over previous
import functools

import jax
import jax.numpy as jnp
from jax import lax
from jax.experimental import pallas as pl
from jax.experimental.pallas import tpu as pltpu
from jax.experimental.pallas import tpu_sc as plsc

_T, _BCOL = 200, 4096
_V, _D = 1_000_000, 32

_info = plsc.get_sparse_core_info()
_NC, _NS = _info.num_cores, _info.num_subcores
_NW = _NC * _NS
_NTILES = (_T // 8) * (_BCOL // 128)
_TPW = _NTILES // _NW
_OUT_WORDS = _T * _BCOL * _D

_mesh = plsc.VectorSubcoreMesh(core_axis_name="c", subcore_axis_name="s")


@functools.partial(
    pl.kernel,
    mesh=_mesh,
    out_type=jax.ShapeDtypeStruct((_OUT_WORDS,), jnp.float32),
    scratch_types=[
        pltpu.VMEM((2, 512), jnp.int32),
        pltpu.VMEM((2, 512, _D), jnp.float32),
        pltpu.VMEM((2, 16384), jnp.float32),
        pltpu.SemaphoreType.DMA,
        pltpu.SemaphoreType.DMA,
        pltpu.SemaphoreType.DMA,
    ],
    compiler_params=pltpu.CompilerParams(
        use_tc_tiling_on_sc=False, needs_layout_passes=False
    ),
)
def _emb_lookup(idx_hbm, table_hbm, out_hbm, idx_v, rows_v, obuf, isem, gsem, osem):
    wid = lax.axis_index("s") * _NC + lax.axis_index("c")
    base = wid * _TPW
    iota16 = lax.iota(jnp.int32, 16)
    c128iota = iota16 * 128
    c2048 = jnp.full((16,), 2048, jnp.int32)

    def issue_idx(k, h, p):
        pltpu.async_copy(idx_hbm.at[k, pl.ds(h * 512, 512)], idx_v.at[p], isem)

    def wait_idx(p):
        pltpu.make_async_copy(idx_hbm.at[0, pl.ds(0, 512)], idx_v.at[p], isem).wait()

    def issue_gather(p):
        pltpu.async_copy(table_hbm.at[idx_v.at[p]], rows_v.at[p], gsem)

    def wait_gather(p):
        pltpu.make_async_copy(
            table_hbm.at[pl.ds(0, 512)], rows_v.at[p], gsem
        ).wait()

    def drain_out(p):
        pltpu.make_async_copy(
            out_hbm.at[pl.ds(0, 16384)], obuf.at[p], osem
        ).wait()

    def transpose_and_store(p, R, C, h):
        rows = rows_v.at[p]
        ob = obuf.at[p]
        cvec0 = iota16
        cvec1 = iota16 + 16

        def qbody(q, carry):
            qb = q * 128
            ob_q = q * 4096
            for tc0 in range(128):
                tcv = tc0 + iota16
                if tc0 > 112:
                    tcv = tcv & 127
                rvec = qb + tcv
                ovec = c128iota + tcv
                for dh, cvec in ((0, cvec0), (16, cvec1)):
                    v = plsc.load_gather(rows, [rvec, cvec])
                    plsc.store_scatter(ob, [ob_q + dh * 128 + ovec], v)
            return carry

        lax.fori_loop(0, 4, qbody, 0)
        for q in range(4):
            for Rd in range(4):
                dst = (8 * R + 4 * h + q) * 131072 + Rd * 32768 + C * 1024
                pltpu.async_copy(
                    ob.at[pl.ds((q * 4 + Rd) * 1024, 1024)],
                    out_hbm.at[pl.ds(dst, 1024)],
                    osem,
                )

    pltpu.sync_copy(idx_hbm.at[base, pl.ds(0, 512)], idx_v.at[0])
    issue_gather(0)
    issue_idx(base, 1, 1)

    def body(i2, carry):
        k = base + i2
        R = k >> 5
        C = k & 31
        kn = k + 1
        @pl.when(i2 > 0)
        def _():
            drain_out(0)

        wait_gather(0)
        wait_idx(1)
        issue_gather(1)

        @pl.when(i2 < _TPW - 1)
        def _():
            issue_idx(kn, 0, 0)

        transpose_and_store(0, R, C, 0)

        @pl.when(i2 > 0)
        def _():
            drain_out(1)

        wait_gather(1)

        @pl.when(i2 < _TPW - 1)
        def _():
            wait_idx(0)
            issue_gather(0)
            issue_idx(kn, 1, 1)

        transpose_and_store(1, R, C, 1)
        return carry

    lax.fori_loop(0, _TPW, body, 0)
    drain_out(0)
    drain_out(1)


@functools.partial(
    pl.kernel,
    mesh=_mesh,
    out_type=jax.ShapeDtypeStruct((_V * _D,), jnp.float32),
    scratch_types=[
        pltpu.VMEM((32, 128), jnp.float32),
        pltpu.VMEM((32, 128), jnp.float32),
        pltpu.VMEM((4096,), jnp.float32),
        pltpu.VMEM((4096,), jnp.float32),
        pltpu.SemaphoreType.DMA,
        pltpu.SemaphoreType.DMA,
    ],
    compiler_params=pltpu.CompilerParams(
        use_tc_tiling_on_sc=True, needs_layout_passes=False
    ),
)
def _detr(tT_hbm, out2_hbm, blk0, blk1, tb0, tb1, bsem, osem):
    wid = lax.axis_index("s") * _NC + lax.axis_index("c")
    iota16 = lax.iota(jnp.int32, 16)
    cvec0 = iota16
    cvec1 = iota16 + 16
    blk = (blk0, blk1)
    tb = (tb0, tb1)
    _NCHUNK = (_V + 127) // 128

    def issue_blk(c, W, p):
        pltpu.async_copy(
            tT_hbm.at[pl.ds(0, 32), pl.ds(pl.multiple_of(128 * c, 128), W)],
            blk[p].at[pl.ds(0, 32), pl.ds(0, W)],
            bsem,
        )

    def wait_blk(W, p):
        pltpu.make_async_copy(
            tT_hbm.at[pl.ds(0, 32), pl.ds(0, W)],
            blk[p].at[pl.ds(0, 32), pl.ds(0, W)],
            bsem,
        ).wait()

    def drain_write(p):
        pltpu.make_async_copy(out2_hbm.at[pl.ds(0, 4096)], tb[p], osem).wait()

    def transpose(p, W):
        for tc0 in range(W):
            tcv = tc0 + iota16
            if tc0 > W - 16:
                tcv = tcv & (W - 1)
            for cvec in (cvec0, cvec1):
                v = plsc.load_gather(blk[p], [cvec, tcv])
                plsc.store_scatter(tb[p], [(tcv << 5) + cvec], v)

    def issue_write(c, W, p):
        pltpu.async_copy(
            tb[p].at[pl.ds(0, W * 32)],
            out2_hbm.at[pl.ds(pl.multiple_of(4096 * c, 8), W * 32)],
            osem,
        )

    issue_blk(wid, 128, 0)
    issue_blk(wid + 32, 128, 1)

    def body(m, carry):
        n0 = 2 * m
        c0 = wid + 32 * n0
        @pl.when(m > 0)
        def _():
            drain_write(0)

        wait_blk(128, 0)
        transpose(0, 128)
        issue_write(c0, 128, 0)

        @pl.when(n0 + 2 <= 243)
        def _():
            issue_blk(c0 + 64, 128, 0)

        @pl.when(m > 0)
        def _():
            drain_write(1)

        wait_blk(128, 1)
        transpose(1, 128)
        issue_write(c0 + 32, 128, 1)

        @pl.when(n0 + 3 <= 243)
        def _():
            issue_blk(c0 + 96, 128, 1)
        return carry

    lax.fori_loop(0, 122, body, 0)
    drain_write(0)
    drain_write(1)

    c_tail = wid + 32 * 244

    @pl.when(c_tail < _NCHUNK - 1)
    def _():
        pltpu.sync_copy(
            tT_hbm.at[pl.ds(0, 32), pl.ds(pl.multiple_of(128 * c_tail, 128), 128)],
            blk[0],
        )
        transpose(0, 128)
        pltpu.sync_copy(
            tb[0], out2_hbm.at[pl.ds(pl.multiple_of(4096 * c_tail, 8), 4096)]
        )

    @pl.when(c_tail == _NCHUNK - 1)
    def _():
        pltpu.sync_copy(
            tT_hbm.at[pl.ds(0, 32), pl.ds(pl.multiple_of(128 * c_tail, 128), 128)],
            blk[1],
        )
        transpose(1, 64)
        pltpu.sync_copy(
            tb[1].at[pl.ds(0, 2048)],
            out2_hbm.at[pl.ds(pl.multiple_of(4096 * c_tail, 8), 2048)],
        )


def kernel(inputs, embedding_weight):
    tlin = _detr(embedding_weight.T).reshape(_V, _D)
    idx4 = (
        inputs.astype(jnp.int32)
        .reshape(_T // 8, 8, _BCOL // 128, 128)
        .transpose(0, 2, 1, 3)
        .reshape(_NTILES, 1024)
    )
    out_flat = _emb_lookup(idx4, tlin)
    return (
        out_flat.reshape(_T, 4, _BCOL // 128, 8, 128)
        .transpose(0, 2, 4, 1, 3)
        .reshape(_T, _BCOL, _D)
    )


# speedup vs baseline: 1.4589x; 1.0012x over previous
import functools

import jax
import jax.numpy as jnp
from jax import lax
from jax.experimental import pallas as pl
from jax.experimental.pallas import tpu as pltpu
from jax.experimental.pallas import tpu_sc as plsc

_T, _BCOL = 200, 4096
_V, _D = 1_000_000, 32

_info = plsc.get_sparse_core_info()
_NC, _NS = _info.num_cores, _info.num_subcores
_NW = _NC * _NS
_NTILES = (_T // 8) * (_BCOL // 128)
_TPW = _NTILES // _NW
_OUT_WORDS = _T * _BCOL * _D

_mesh = plsc.VectorSubcoreMesh(core_axis_name="c", subcore_axis_name="s")


@functools.partial(
    pl.kernel,
    mesh=_mesh,
    out_type=jax.ShapeDtypeStruct((_OUT_WORDS,), jnp.float32),
    scratch_types=[
        pltpu.VMEM((2, 512), jnp.int32),
        pltpu.VMEM((2, 512, _D), jnp.float32),
        pltpu.VMEM((2, 16384), jnp.float32),
        pltpu.SemaphoreType.DMA,
        pltpu.SemaphoreType.DMA,
        pltpu.SemaphoreType.DMA,
    ],
    compiler_params=pltpu.CompilerParams(
        use_tc_tiling_on_sc=False, needs_layout_passes=False
    ),
)
def _emb_lookup(idx_hbm, table_hbm, out_hbm, idx_v, rows_v, obuf, isem, gsem, osem):
    wid = lax.axis_index("s") * _NC + lax.axis_index("c")
    base = wid * _TPW
    iota16 = lax.iota(jnp.int32, 16)
    c128iota = iota16 * 128
    c2048 = jnp.full((16,), 2048, jnp.int32)

    def issue_idx(k, h, p):
        pltpu.async_copy(idx_hbm.at[k, pl.ds(h * 512, 512)], idx_v.at[p], isem)

    def wait_idx(p):
        pltpu.make_async_copy(idx_hbm.at[0, pl.ds(0, 512)], idx_v.at[p], isem).wait()

    def issue_gather(p):
        pltpu.async_copy(table_hbm.at[idx_v.at[p]], rows_v.at[p], gsem)

    def wait_gather(p):
        pltpu.make_async_copy(
            table_hbm.at[pl.ds(0, 512)], rows_v.at[p], gsem
        ).wait()

    def drain_out(p):
        pltpu.make_async_copy(
            out_hbm.at[pl.ds(0, 16384)], obuf.at[p], osem
        ).wait()

    def transpose_and_store(p, R, C, h):
        rows = rows_v.at[p]
        ob = obuf.at[p]
        cvec0 = iota16
        cvec1 = iota16 + 16

        def qbody(q, carry):
            qb = q * 128
            ob_q = q * 4096
            for tc0 in range(128):
                tcv = tc0 + iota16
                if tc0 > 112:
                    tcv = tcv & 127
                rvec = qb + tcv
                ovec = c128iota + tcv
                for dh, cvec in ((0, cvec0), (16, cvec1)):
                    v = plsc.load_gather(rows, [rvec, cvec])
                    plsc.store_scatter(ob, [ob_q + dh * 128 + ovec], v)
            return carry

        lax.fori_loop(0, 4, qbody, 0)
        for q in range(4):
            for Rd in range(4):
                dst = (8 * R + 4 * h + q) * 131072 + Rd * 32768 + C * 1024
                pltpu.async_copy(
                    ob.at[pl.ds((q * 4 + Rd) * 1024, 1024)],
                    out_hbm.at[pl.ds(dst, 1024)],
                    osem,
                )

    pltpu.sync_copy(idx_hbm.at[base, pl.ds(0, 512)], idx_v.at[0])
    issue_gather(0)
    issue_idx(base, 1, 1)

    def body(i2, carry):
        k = base + i2
        R = k >> 5
        C = k & 31
        kn = k + 1
        @pl.when(i2 > 0)
        def _():
            drain_out(0)

        wait_gather(0)
        wait_idx(1)
        issue_gather(1)

        @pl.when(i2 < _TPW - 1)
        def _():
            issue_idx(kn, 0, 0)

        transpose_and_store(0, R, C, 0)

        @pl.when(i2 > 0)
        def _():
            drain_out(1)

        wait_gather(1)

        @pl.when(i2 < _TPW - 1)
        def _():
            wait_idx(0)
            issue_gather(0)
            issue_idx(kn, 1, 1)

        transpose_and_store(1, R, C, 1)
        return carry

    lax.fori_loop(0, _TPW, body, 0)
    drain_out(0)
    drain_out(1)


def kernel(inputs, embedding_weight):
    idx4 = (
        inputs.astype(jnp.int32)
        .reshape(_T // 8, 8, _BCOL // 128, 128)
        .transpose(0, 2, 1, 3)
        .reshape(_NTILES, 1024)
    )
    out_flat = _emb_lookup(idx4, embedding_weight)
    return (
        out_flat.reshape(_T, 4, _BCOL // 128, 8, 128)
        .transpose(0, 2, 4, 1, 3)
        .reshape(_T, _BCOL, _D)
    )


# speedup vs baseline: 1.4596x; 1.0005x over previous
import functools

import jax
import jax.numpy as jnp
from jax import lax
from jax.experimental import pallas as pl
from jax.experimental.pallas import tpu as pltpu
from jax.experimental.pallas import tpu_sc as plsc

_T, _BCOL = 200, 4096
_V, _D = 1_000_000, 32

_info = plsc.get_sparse_core_info()
_NC, _NS = _info.num_cores, _info.num_subcores
_NW = _NC * _NS
_NTILES = (_T // 8) * (_BCOL // 128)
_TPW = _NTILES // _NW
_OUT_WORDS = _T * _BCOL * _D

_mesh = plsc.VectorSubcoreMesh(core_axis_name="c", subcore_axis_name="s")


@functools.partial(
    pl.kernel,
    mesh=_mesh,
    out_type=jax.ShapeDtypeStruct((_OUT_WORDS,), jnp.float32),
    scratch_types=[
        pltpu.VMEM((2, 512), jnp.int32),
        pltpu.VMEM((2, 512, _D), jnp.float32),
        pltpu.VMEM((2, 16384), jnp.float32),
        pltpu.SemaphoreType.DMA,
        pltpu.SemaphoreType.DMA,
        pltpu.SemaphoreType.DMA,
    ],
    compiler_params=pltpu.CompilerParams(
        use_tc_tiling_on_sc=False, needs_layout_passes=False
    ),
)
def _emb_lookup(idx_hbm, table_hbm, out_hbm, idx_v, rows_v, obuf, isem, gsem, osem):
    wid = lax.axis_index("s") * _NC + lax.axis_index("c")
    base = wid * _TPW
    iota16 = lax.iota(jnp.int32, 16)
    c128iota = iota16 * 128
    c2048 = jnp.full((16,), 2048, jnp.int32)

    def issue_idx(k, h, p):
        pltpu.async_copy(idx_hbm.at[k, pl.ds(h * 512, 512)], idx_v.at[p], isem)

    def wait_idx(p):
        pltpu.make_async_copy(idx_hbm.at[0, pl.ds(0, 512)], idx_v.at[p], isem).wait()

    def issue_gather(p):
        pltpu.async_copy(table_hbm.at[idx_v.at[p]], rows_v.at[p], gsem)

    def wait_gather(p):
        pltpu.make_async_copy(
            table_hbm.at[pl.ds(0, 512)], rows_v.at[p], gsem
        ).wait()

    def drain_out(p):
        pltpu.make_async_copy(
            out_hbm.at[pl.ds(0, 16384)], obuf.at[p], osem
        ).wait()

    def transpose_and_store(p, R, C, h):
        rows = rows_v.at[p]
        ob = obuf.at[p]
        cvec0 = iota16
        cvec1 = iota16 + 16

        @plsc.parallel_loop(0, 4)
        def _(q):
            qb = q * 128
            ob_q = q * 4096
            for tc0 in range(128):
                tcv = tc0 + iota16
                if tc0 > 112:
                    tcv = tcv & 127
                rvec = qb + tcv
                ovec = c128iota + tcv
                for dh, cvec in ((0, cvec0), (16, cvec1)):
                    v = plsc.load_gather(rows, [rvec, cvec])
                    plsc.store_scatter(ob, [ob_q + dh * 128 + ovec], v)
        for q in range(4):
            for Rd in range(4):
                dst = (8 * R + 4 * h + q) * 131072 + Rd * 32768 + C * 1024
                pltpu.async_copy(
                    ob.at[pl.ds((q * 4 + Rd) * 1024, 1024)],
                    out_hbm.at[pl.ds(dst, 1024)],
                    osem,
                )

    pltpu.sync_copy(idx_hbm.at[base, pl.ds(0, 512)], idx_v.at[0])
    issue_gather(0)
    issue_idx(base, 1, 1)

    def body(i2, carry):
        k = base + i2
        R = k >> 5
        C = k & 31
        kn = k + 1
        @pl.when(i2 > 0)
        def _():
            drain_out(0)

        wait_gather(0)
        wait_idx(1)
        issue_gather(1)

        @pl.when(i2 < _TPW - 1)
        def _():
            issue_idx(kn, 0, 0)

        transpose_and_store(0, R, C, 0)

        @pl.when(i2 > 0)
        def _():
            drain_out(1)

        wait_gather(1)

        @pl.when(i2 < _TPW - 1)
        def _():
            wait_idx(0)
            issue_gather(0)
            issue_idx(kn, 1, 1)

        transpose_and_store(1, R, C, 1)
        return carry

    lax.fori_loop(0, _TPW, body, 0)
    drain_out(0)
    drain_out(1)


def kernel(inputs, embedding_weight):
    idx4 = (
        inputs.astype(jnp.int32)
        .reshape(_T // 8, 8, _BCOL // 128, 128)
        .transpose(0, 2, 1, 3)
        .reshape(_NTILES, 1024)
    )
    out_flat = _emb_lookup(idx4, embedding_weight)
    return (
        out_flat.reshape(_T, 4, _BCOL // 128, 8, 128)
        .transpose(0, 2, 4, 1, 3)
        .reshape(_T, _BCOL, _D)
    )
